# Initial kernel scaffold; baseline (speedup 1.0000x reference)
import jax
import jax.numpy as jnp
from jax.experimental import pallas as pl


def kernel(x, edge_index, edge_attr, mask_node_indices, prelu_a, W_enc, emb1, emb2, emb3, emb4, W1, b1, W2, b2):
    raise NotImplementedError("write your pallas kernel here")


# speedup vs baseline: 5.3913x; 5.3913x over previous
import functools

import jax
import jax.numpy as jnp
from jax import lax
from jax.experimental import pallas as pl
from jax.experimental.pallas import tpu as pltpu
from jax.experimental.pallas import tpu_sc as plsc

N = 10000
E = 160000
D = 256
HALF = 128
OUT = 256

EP = 163840
NPAD = EP - E
ET = EP // 16
EW = EP // 32
CHUNK = 128
GMAIN = ET // CHUNK
GCNT = EW // CHUNK
GRP = 8
ROWS_T = 10112 // 16
NR = 10112

MB = 1536
RB = 1000
EB = EP // 64


def _h_body(x_ref, w_ref, a_ref, m_ref, out_ref):
    i = pl.program_id(0)
    xb = x_ref[...]
    a = a_ref[0, 0]
    h = jnp.where(xb > 0, xb, a * xb)
    h = lax.dot_general(h, w_ref[...], (((1,), (1,)), ((), ())),
                        preferred_element_type=jnp.float32)
    rows = i * RB + lax.broadcasted_iota(jnp.int32, (RB, 1), 0)
    masked = jnp.any(rows == m_ref[...], axis=1)
    h = jnp.where(masked[:, None], 0.0, h)
    out_ref[0] = h[:, :HALF]
    out_ref[1] = h[:, HALF:]


def _onehot_body(ea_ref, out_ref):
    i = pl.program_id(0)
    a = ea_ref[...]
    j = lax.broadcasted_iota(jnp.int32, (EB, HALF), 1)
    oh = ((j == a[:, 0:1]).astype(jnp.float32)
          + (j == 5 + a[:, 1:2]).astype(jnp.float32)
          + (j == 8 + a[:, 2:3]).astype(jnp.float32)
          + (j == 11 + a[:, 3:4]).astype(jnp.float32))
    rows = i * EB + lax.broadcasted_iota(jnp.int32, (EB, 1), 0)
    out_ref[...] = jnp.where(rows < E, oh, 0.0)


def _mlp_body(h_ref, seg_ref, cnt_ref, cemb_ref, w1_ref, b1_ref, w2_ref,
              b2_ref, out_ref):
    h = jnp.concatenate([h_ref[0], h_ref[1]], axis=1)
    seg = jnp.concatenate([seg_ref[0], seg_ref[1]], axis=1)
    cnt = cnt_ref[0][:, :16] + cnt_ref[1][:, :16]
    cemb = cemb_ref[...]
    const = cemb[4] + cemb[5] + cemb[8] + cemb[11]
    aggr = (h + seg + const[None, :]
            + jnp.dot(cnt, cemb, preferred_element_type=jnp.float32))
    hid = lax.dot_general(aggr, w1_ref[...], (((1,), (1,)), ((), ())),
                          preferred_element_type=jnp.float32)
    hid = jnp.maximum(hid + b1_ref[...], 0.0)
    o = lax.dot_general(hid, w2_ref[...], (((1,), (1,)), ((), ())),
                        preferred_element_type=jnp.float32)
    out_ref[...] = o + b2_ref[...]


def _sc_main_body(h_hbm, src_hbm, dst_hbm, z128_hbm, seg_hbm,
                  src_c, dst_c, gbuf, aggr_sh, sem):
    c = lax.axis_index("c")
    s = lax.axis_index("s")

    pltpu.sync_copy(z128_hbm, gbuf)
    for t, nr in ((0, CHUNK), (1, CHUNK), (2, CHUNK), (3, CHUNK),
                  (4, ROWS_T - 4 * CHUNK)):
        pltpu.sync_copy(gbuf.at[pl.ds(0, nr)],
                        aggr_sh.at[pl.ds(s * ROWS_T + t * CHUNK, nr)])

    plsc.subcore_barrier()

    def main_body(g, cy):
        pltpu.sync_copy(src_hbm.at[c * (EP // CHUNK) + s * GMAIN + g], src_c)
        pltpu.sync_copy(dst_hbm.at[s * GMAIN + g], dst_c)
        pltpu.async_copy(h_hbm.at[src_c], gbuf, sem).wait()
        pltpu.sync_copy(gbuf, aggr_sh.at[dst_c], add=True)
        return cy

    lax.fori_loop(0, GMAIN, main_body, 0)

    plsc.subcore_barrier()

    for t, nr in ((0, CHUNK), (1, CHUNK), (2, CHUNK), (3, CHUNK),
                  (4, ROWS_T - 4 * CHUNK)):
        r0 = s * ROWS_T + t * CHUNK
        pltpu.sync_copy(aggr_sh.at[pl.ds(r0, nr)], gbuf.at[pl.ds(0, nr)])
        pltpu.sync_copy(gbuf.at[pl.ds(0, nr)],
                        seg_hbm.at[pl.ds(c * NR + r0, nr)])


def _sc_cnt_body(dst_hbm, oh_hbm, z128_hbm, segdep_hbm, cnt_hbm,
                 dst_c, obuf, cnt_sh, sem):
    del segdep_hbm
    c = lax.axis_index("c")
    s = lax.axis_index("s")
    wid = c * 16 + s

    pltpu.sync_copy(z128_hbm, obuf)
    for t, nr in ((0, CHUNK), (1, CHUNK), (2, CHUNK), (3, CHUNK),
                  (4, ROWS_T - 4 * CHUNK)):
        pltpu.sync_copy(obuf.at[pl.ds(0, nr)],
                        cnt_sh.at[pl.ds(s * ROWS_T + t * CHUNK, nr)])

    plsc.subcore_barrier()

    def cnt_body(g, cy):
        pltpu.sync_copy(dst_hbm.at[wid * GCNT + g], dst_c)
        pltpu.sync_copy(
            oh_hbm.at[pl.ds((wid * GCNT + g) * CHUNK, CHUNK)], obuf)
        pltpu.sync_copy(obuf, cnt_sh.at[dst_c], add=True)
        return cy

    lax.fori_loop(0, GCNT, cnt_body, 0)

    plsc.subcore_barrier()

    for t, nr in ((0, CHUNK), (1, CHUNK), (2, CHUNK), (3, CHUNK),
                  (4, ROWS_T - 4 * CHUNK)):
        r0 = s * ROWS_T + t * CHUNK
        pltpu.sync_copy(cnt_sh.at[pl.ds(r0, nr)], obuf.at[pl.ds(0, nr)])
        pltpu.sync_copy(obuf.at[pl.ds(0, nr)],
                        cnt_hbm.at[pl.ds(c * NR + r0, nr)])


_sc_main = functools.partial(
    pl.kernel,
    out_type=jax.ShapeDtypeStruct((2 * NR, HALF), jnp.float32),
    mesh=plsc.VectorSubcoreMesh(core_axis_name="c", subcore_axis_name="s"),
    scratch_types=[
        pltpu.VMEM((CHUNK,), jnp.int32),
        pltpu.VMEM((CHUNK,), jnp.int32),
        pltpu.VMEM((CHUNK, HALF), jnp.float32),
        pltpu.VMEM_SHARED((NR, HALF), jnp.float32),
        pltpu.SemaphoreType.DMA,
    ],
)(_sc_main_body)

_sc_cnt = functools.partial(
    pl.kernel,
    out_type=jax.ShapeDtypeStruct((2 * NR, HALF), jnp.float32),
    mesh=plsc.VectorSubcoreMesh(core_axis_name="c", subcore_axis_name="s"),
    scratch_types=[
        pltpu.VMEM((CHUNK,), jnp.int32),
        pltpu.VMEM((CHUNK, HALF), jnp.float32),
        pltpu.VMEM_SHARED((NR, HALF), jnp.float32),
        pltpu.SemaphoreType.DMA,
    ],
)(_sc_cnt_body)


@jax.jit
def kernel(x, edge_index, edge_attr, mask_node_indices, prelu_a, W_enc,
           emb1, emb2, emb3, emb4, W1, b1, W2, b2):
    mpad = jnp.pad(mask_node_indices, (0, MB - mask_node_indices.shape[0]),
                   constant_values=-1).reshape(1, MB)
    h_split = pl.pallas_call(
        _h_body,
        grid=(N // RB,),
        in_specs=[
            pl.BlockSpec((RB, D), lambda i: (i, 0)),
            pl.BlockSpec((D, D), lambda i: (0, 0)),
            pl.BlockSpec((1, 1), lambda i: (0, 0)),
            pl.BlockSpec((1, MB), lambda i: (0, 0)),
        ],
        out_specs=pl.BlockSpec((2, RB, HALF), lambda i: (0, i, 0)),
        out_shape=jax.ShapeDtypeStruct((2, N, HALF), jnp.float32),
    )(x, W_enc, prelu_a.reshape(1, 1), mpad)

    ea_p = jnp.pad(edge_attr, ((0, NPAD), (0, 0)))
    onehot = pl.pallas_call(
        _onehot_body,
        grid=(64,),
        in_specs=[pl.BlockSpec((EB, 4), lambda i: (i, 0))],
        out_specs=pl.BlockSpec((EB, HALF), lambda i: (i, 0)),
        out_shape=jax.ShapeDtypeStruct((EP, HALF), jnp.float32),
    )(ea_p)

    src_p = jnp.concatenate([edge_index[0], jnp.zeros((NPAD,), jnp.int32)])
    src2d = jnp.concatenate([src_p, src_p + N]).reshape(-1, CHUNK)
    dst2d = jnp.concatenate(
        [edge_index[1], jnp.full((NPAD,), N, jnp.int32)]).reshape(-1, CHUNK)
    h_flat = h_split.reshape(2 * N, HALF)
    z128 = jnp.zeros((CHUNK, HALF), jnp.float32)
    seg_flat = _sc_main(h_flat, src2d, dst2d, z128)
    cnt_flat = _sc_cnt(dst2d, onehot, z128, seg_flat)
    seg = seg_flat.reshape(2, NR, HALF)[:, :N, :]
    cnt = cnt_flat.reshape(2, NR, HALF)[:, :N, :]

    cemb = jnp.concatenate(
        [emb1, emb2, emb3, emb4, jnp.zeros((2, D), jnp.float32)], axis=0)
    out = pl.pallas_call(
        _mlp_body,
        grid=(N // RB,),
        in_specs=[
            pl.BlockSpec((2, RB, HALF), lambda i: (0, i, 0)),
            pl.BlockSpec((2, RB, HALF), lambda i: (0, i, 0)),
            pl.BlockSpec((2, RB, HALF), lambda i: (0, i, 0)),
            pl.BlockSpec((16, D), lambda i: (0, 0)),
            pl.BlockSpec((2 * D, D), lambda i: (0, 0)),
            pl.BlockSpec((1, 2 * D), lambda i: (0, 0)),
            pl.BlockSpec((OUT, 2 * D), lambda i: (0, 0)),
            pl.BlockSpec((1, OUT), lambda i: (0, 0)),
        ],
        out_specs=pl.BlockSpec((RB, OUT), lambda i: (i, 0)),
        out_shape=jax.ShapeDtypeStruct((N, OUT), jnp.float32),
    )(h_split, seg, cnt, cemb, W1, b1.reshape(1, 2 * D), W2,
      b2.reshape(1, OUT))
    return out


# speedup vs baseline: 5.9032x; 1.0949x over previous
import functools

import jax
import jax.numpy as jnp
from jax import lax
from jax.experimental import pallas as pl
from jax.experimental.pallas import tpu as pltpu
from jax.experimental.pallas import tpu_sc as plsc

N = 10000
E = 160000
D = 256
HALF = 128
OUT = 256

EP = 163840
NPAD = EP - E
ET = EP // 16
EW = EP // 32
CHUNK = 128
GMAIN = ET // CHUNK
GCNT = EW // CHUNK
GRP = 8
ROWS_T = 10112 // 16
NR = 10112

MB = 1536
RB = 1000
EB = EP // 64


def _h_body(x_ref, w_ref, a_ref, m_ref, out_ref):
    i = pl.program_id(0)
    xb = x_ref[...]
    a = a_ref[0, 0]
    h = jnp.where(xb > 0, xb, a * xb)
    h = lax.dot_general(h, w_ref[...], (((1,), (1,)), ((), ())),
                        preferred_element_type=jnp.float32)
    rows = i * RB + lax.broadcasted_iota(jnp.int32, (RB, 1), 0)
    masked = jnp.any(rows == m_ref[...], axis=1)
    h = jnp.where(masked[:, None], 0.0, h)
    out_ref[0] = h[:, :HALF]
    out_ref[1] = h[:, HALF:]


def _onehot_body(ea_ref, out_ref):
    i = pl.program_id(0)
    a = ea_ref[...]
    j = lax.broadcasted_iota(jnp.int32, (EB, HALF), 1)
    oh = ((j == a[:, 0:1]).astype(jnp.float32)
          + (j == 5 + a[:, 1:2]).astype(jnp.float32)
          + (j == 8 + a[:, 2:3]).astype(jnp.float32)
          + (j == 11 + a[:, 3:4]).astype(jnp.float32))
    rows = i * EB + lax.broadcasted_iota(jnp.int32, (EB, 1), 0)
    out_ref[...] = jnp.where(rows < E, oh, 0.0)


def _mlp_body(h_ref, seg_ref, cnt_ref, cemb_ref, w1_ref, b1_ref, w2_ref,
              b2_ref, out_ref):
    h = jnp.concatenate([h_ref[0], h_ref[1]], axis=1)
    seg = jnp.concatenate([seg_ref[0], seg_ref[1]], axis=1)
    cnt = cnt_ref[0][:, :16] + cnt_ref[1][:, :16]
    cemb = cemb_ref[...]
    const = cemb[4] + cemb[5] + cemb[8] + cemb[11]
    aggr = (h + seg + const[None, :]
            + jnp.dot(cnt, cemb, preferred_element_type=jnp.float32))
    hid = lax.dot_general(aggr, w1_ref[...], (((1,), (1,)), ((), ())),
                          preferred_element_type=jnp.float32)
    hid = jnp.maximum(hid + b1_ref[...], 0.0)
    o = lax.dot_general(hid, w2_ref[...], (((1,), (1,)), ((), ())),
                        preferred_element_type=jnp.float32)
    out_ref[...] = o + b2_ref[...]


def _sc_main_body(h_hbm, sd_hbm, z128_hbm, seg_hbm,
                  sd_c, gbufa, gbufb, aggr_sh, sema, semb):
    c = lax.axis_index("c")
    s = lax.axis_index("s")

    pltpu.sync_copy(z128_hbm, gbufa)
    for t, nr in ((0, CHUNK), (1, CHUNK), (2, CHUNK), (3, CHUNK),
                  (4, ROWS_T - 4 * CHUNK)):
        pltpu.sync_copy(gbufa.at[pl.ds(0, nr)],
                        aggr_sh.at[pl.ds(s * ROWS_T + t * CHUNK, nr)])

    plsc.subcore_barrier()

    def main_body(g, cy):
        pair = c * (EP // CHUNK) + s * GMAIN + 2 * g
        pltpu.sync_copy(sd_hbm.at[pl.ds(2 * pair, 4)], sd_c)
        cpa = pltpu.async_copy(h_hbm.at[sd_c.at[0]], gbufa, sema)
        cpb = pltpu.async_copy(h_hbm.at[sd_c.at[2]], gbufb, semb)
        cpa.wait()
        pltpu.sync_copy(gbufa, aggr_sh.at[sd_c.at[1]], add=True)
        cpb.wait()
        pltpu.sync_copy(gbufb, aggr_sh.at[sd_c.at[3]], add=True)
        return cy

    lax.fori_loop(0, GMAIN // 2, main_body, 0)

    plsc.subcore_barrier()

    for t, nr in ((0, CHUNK), (1, CHUNK), (2, CHUNK), (3, CHUNK),
                  (4, ROWS_T - 4 * CHUNK)):
        r0 = s * ROWS_T + t * CHUNK
        pltpu.sync_copy(aggr_sh.at[pl.ds(r0, nr)], gbufa.at[pl.ds(0, nr)])
        pltpu.sync_copy(gbufa.at[pl.ds(0, nr)],
                        seg_hbm.at[pl.ds(c * NR + r0, nr)])


def _sc_cnt_body(dst_hbm, oh_hbm, z128_hbm, segdep_hbm, cnt_hbm,
                 dst_c, obuf, cnt_sh, sem):
    del segdep_hbm
    c = lax.axis_index("c")
    s = lax.axis_index("s")
    wid = c * 16 + s

    pltpu.sync_copy(z128_hbm, obuf)
    for t, nr in ((0, CHUNK), (1, CHUNK), (2, CHUNK), (3, CHUNK),
                  (4, ROWS_T - 4 * CHUNK)):
        pltpu.sync_copy(obuf.at[pl.ds(0, nr)],
                        cnt_sh.at[pl.ds(s * ROWS_T + t * CHUNK, nr)])

    plsc.subcore_barrier()

    def cnt_body(g, cy):
        pltpu.sync_copy(dst_hbm.at[wid * GCNT + g], dst_c)
        pltpu.sync_copy(
            oh_hbm.at[pl.ds((wid * GCNT + g) * CHUNK, CHUNK)], obuf)
        pltpu.sync_copy(obuf, cnt_sh.at[dst_c], add=True)
        return cy

    lax.fori_loop(0, GCNT, cnt_body, 0)

    plsc.subcore_barrier()

    for t, nr in ((0, CHUNK), (1, CHUNK), (2, CHUNK), (3, CHUNK),
                  (4, ROWS_T - 4 * CHUNK)):
        r0 = s * ROWS_T + t * CHUNK
        pltpu.sync_copy(cnt_sh.at[pl.ds(r0, nr)], obuf.at[pl.ds(0, nr)])
        pltpu.sync_copy(obuf.at[pl.ds(0, nr)],
                        cnt_hbm.at[pl.ds(c * NR + r0, nr)])


_sc_main = functools.partial(
    pl.kernel,
    out_type=jax.ShapeDtypeStruct((2 * NR, HALF), jnp.float32),
    mesh=plsc.VectorSubcoreMesh(core_axis_name="c", subcore_axis_name="s"),
    scratch_types=[
        pltpu.VMEM((4, CHUNK), jnp.int32),
        pltpu.VMEM((CHUNK, HALF), jnp.float32),
        pltpu.VMEM((CHUNK, HALF), jnp.float32),
        pltpu.VMEM_SHARED((NR, HALF), jnp.float32),
        pltpu.SemaphoreType.DMA,
        pltpu.SemaphoreType.DMA,
    ],
)(_sc_main_body)

_sc_cnt = functools.partial(
    pl.kernel,
    out_type=jax.ShapeDtypeStruct((2 * NR, HALF), jnp.float32),
    mesh=plsc.VectorSubcoreMesh(core_axis_name="c", subcore_axis_name="s"),
    scratch_types=[
        pltpu.VMEM((CHUNK,), jnp.int32),
        pltpu.VMEM((CHUNK, HALF), jnp.float32),
        pltpu.VMEM_SHARED((NR, HALF), jnp.float32),
        pltpu.SemaphoreType.DMA,
    ],
)(_sc_cnt_body)


@jax.jit
def kernel(x, edge_index, edge_attr, mask_node_indices, prelu_a, W_enc,
           emb1, emb2, emb3, emb4, W1, b1, W2, b2):
    mpad = jnp.pad(mask_node_indices, (0, MB - mask_node_indices.shape[0]),
                   constant_values=-1).reshape(1, MB)
    h_split = pl.pallas_call(
        _h_body,
        grid=(N // RB,),
        in_specs=[
            pl.BlockSpec((RB, D), lambda i: (i, 0)),
            pl.BlockSpec((D, D), lambda i: (0, 0)),
            pl.BlockSpec((1, 1), lambda i: (0, 0)),
            pl.BlockSpec((1, MB), lambda i: (0, 0)),
        ],
        out_specs=pl.BlockSpec((2, RB, HALF), lambda i: (0, i, 0)),
        out_shape=jax.ShapeDtypeStruct((2, N, HALF), jnp.float32),
    )(x, W_enc, prelu_a.reshape(1, 1), mpad)

    ea_p = jnp.pad(edge_attr, ((0, NPAD), (0, 0)))
    onehot = pl.pallas_call(
        _onehot_body,
        grid=(64,),
        in_specs=[pl.BlockSpec((EB, 4), lambda i: (i, 0))],
        out_specs=pl.BlockSpec((EB, HALF), lambda i: (i, 0)),
        out_shape=jax.ShapeDtypeStruct((EP, HALF), jnp.float32),
    )(ea_p)

    src_p = jnp.concatenate([edge_index[0], jnp.zeros((NPAD,), jnp.int32)])
    dst_p = jnp.concatenate(
        [edge_index[1], jnp.full((NPAD,), N, jnp.int32)])
    dst2d = dst_p.reshape(-1, CHUNK)
    src3d = jnp.concatenate(
        [src_p, src_p + N]).reshape(2, EP // CHUNK, CHUNK)
    sd2d = jnp.stack(
        [src3d,
         jnp.broadcast_to(dst2d[None], (2, EP // CHUNK, CHUNK))],
        axis=2).reshape(4 * (EP // CHUNK), CHUNK)
    h_flat = h_split.reshape(2 * N, HALF)
    z128 = jnp.zeros((CHUNK, HALF), jnp.float32)
    seg_flat = _sc_main(h_flat, sd2d, z128)
    cnt_flat = _sc_cnt(dst2d, onehot, z128, seg_flat)
    seg = seg_flat.reshape(2, NR, HALF)[:, :N, :]
    cnt = cnt_flat.reshape(2, NR, HALF)[:, :N, :]

    cemb = jnp.concatenate(
        [emb1, emb2, emb3, emb4, jnp.zeros((2, D), jnp.float32)], axis=0)
    out = pl.pallas_call(
        _mlp_body,
        grid=(N // RB,),
        in_specs=[
            pl.BlockSpec((2, RB, HALF), lambda i: (0, i, 0)),
            pl.BlockSpec((2, RB, HALF), lambda i: (0, i, 0)),
            pl.BlockSpec((2, RB, HALF), lambda i: (0, i, 0)),
            pl.BlockSpec((16, D), lambda i: (0, 0)),
            pl.BlockSpec((2 * D, D), lambda i: (0, 0)),
            pl.BlockSpec((1, 2 * D), lambda i: (0, 0)),
            pl.BlockSpec((OUT, 2 * D), lambda i: (0, 0)),
            pl.BlockSpec((1, OUT), lambda i: (0, 0)),
        ],
        out_specs=pl.BlockSpec((RB, OUT), lambda i: (i, 0)),
        out_shape=jax.ShapeDtypeStruct((N, OUT), jnp.float32),
    )(h_split, seg, cnt, cemb, W1, b1.reshape(1, 2 * D), W2,
      b2.reshape(1, OUT))
    return out


# speedup vs baseline: 5.9308x; 1.0047x over previous
import functools

import jax
import jax.numpy as jnp
from jax import lax
from jax.experimental import pallas as pl
from jax.experimental.pallas import tpu as pltpu
from jax.experimental.pallas import tpu_sc as plsc

N = 10000
E = 160000
D = 256
HALF = 128
OUT = 256

EP = 163840
NPAD = EP - E
ET = EP // 16
EW = EP // 32
CHUNK = 128
GMAIN = ET // CHUNK
GCNT = EW // CHUNK
GRP = 8
ROWS_T = 10112 // 16
NR = 10112

MB = 1536
RB = 1000
EB = EP // 64


def _h_body(x_ref, w_ref, a_ref, m_ref, out_ref):
    i = pl.program_id(0)
    xb = x_ref[...]
    a = a_ref[0, 0]
    h = jnp.where(xb > 0, xb, a * xb)
    h = lax.dot_general(h, w_ref[...], (((1,), (1,)), ((), ())),
                        preferred_element_type=jnp.float32)
    rows = i * RB + lax.broadcasted_iota(jnp.int32, (RB, 1), 0)
    masked = jnp.any(rows == m_ref[...], axis=1)
    h = jnp.where(masked[:, None], 0.0, h)
    out_ref[0] = h[:, :HALF]
    out_ref[1] = h[:, HALF:]


def _onehot_body(ea_ref, out_ref):
    i = pl.program_id(0)
    a = ea_ref[...]
    j = lax.broadcasted_iota(jnp.int32, (EB, HALF), 1)
    oh = ((j == a[:, 0:1]).astype(jnp.float32)
          + (j == 5 + a[:, 1:2]).astype(jnp.float32)
          + (j == 8 + a[:, 2:3]).astype(jnp.float32)
          + (j == 11 + a[:, 3:4]).astype(jnp.float32))
    rows = i * EB + lax.broadcasted_iota(jnp.int32, (EB, 1), 0)
    out_ref[...] = jnp.where(rows < E, oh, 0.0)


def _mlp_body(h_ref, seg_ref, cnt_ref, cemb_ref, w1_ref, b1_ref, w2_ref,
              b2_ref, out_ref):
    h = jnp.concatenate([h_ref[0], h_ref[1]], axis=1)
    seg = jnp.concatenate([seg_ref[0], seg_ref[1]], axis=1)
    cnt = cnt_ref[0][:, :16] + cnt_ref[1][:, :16]
    cemb = cemb_ref[...]
    const = cemb[4] + cemb[5] + cemb[8] + cemb[11]
    aggr = (h + seg + const[None, :]
            + jnp.dot(cnt, cemb, preferred_element_type=jnp.float32))
    hid = lax.dot_general(aggr, w1_ref[...], (((1,), (1,)), ((), ())),
                          preferred_element_type=jnp.float32)
    hid = jnp.maximum(hid + b1_ref[...], 0.0)
    o = lax.dot_general(hid, w2_ref[...], (((1,), (1,)), ((), ())),
                        preferred_element_type=jnp.float32)
    out_ref[...] = o + b2_ref[...]


def _sc_main_body(h_hbm, sd_hbm, z128_hbm, seg_hbm,
                  sd_c, gbufa, gbufb, aggr_sh, sema, semb):
    c = lax.axis_index("c")
    s = lax.axis_index("s")

    pltpu.sync_copy(z128_hbm, gbufa)
    for t, nr in ((0, CHUNK), (1, CHUNK), (2, CHUNK), (3, CHUNK),
                  (4, ROWS_T - 4 * CHUNK)):
        pltpu.sync_copy(gbufa.at[pl.ds(0, nr)],
                        aggr_sh.at[pl.ds(s * ROWS_T + t * CHUNK, nr)])

    plsc.subcore_barrier()

    def main_body(g, cy):
        pair = c * (EP // CHUNK) + s * GMAIN + 2 * g
        pltpu.sync_copy(sd_hbm.at[pl.ds(2 * pair, 4)], sd_c)
        cpa = pltpu.async_copy(h_hbm.at[sd_c.at[0]], gbufa, sema)
        cpb = pltpu.async_copy(h_hbm.at[sd_c.at[2]], gbufb, semb)
        cpa.wait()
        sca = pltpu.async_copy(gbufa, aggr_sh.at[sd_c.at[1]], sema,
                               add=True)
        cpb.wait()
        scb = pltpu.async_copy(gbufb, aggr_sh.at[sd_c.at[3]], semb,
                               add=True)
        sca.wait()
        scb.wait()
        return cy

    lax.fori_loop(0, GMAIN // 2, main_body, 0)

    plsc.subcore_barrier()

    for t, nr in ((0, CHUNK), (1, CHUNK), (2, CHUNK), (3, CHUNK),
                  (4, ROWS_T - 4 * CHUNK)):
        r0 = s * ROWS_T + t * CHUNK
        pltpu.sync_copy(aggr_sh.at[pl.ds(r0, nr)], gbufa.at[pl.ds(0, nr)])
        pltpu.sync_copy(gbufa.at[pl.ds(0, nr)],
                        seg_hbm.at[pl.ds(c * NR + r0, nr)])


def _sc_cnt_body(dst_hbm, oh_hbm, z128_hbm, segdep_hbm, cnt_hbm,
                 dst_c, obuf, cnt_sh, sem):
    del segdep_hbm
    c = lax.axis_index("c")
    s = lax.axis_index("s")
    wid = c * 16 + s

    pltpu.sync_copy(z128_hbm, obuf)
    for t, nr in ((0, CHUNK), (1, CHUNK), (2, CHUNK), (3, CHUNK),
                  (4, ROWS_T - 4 * CHUNK)):
        pltpu.sync_copy(obuf.at[pl.ds(0, nr)],
                        cnt_sh.at[pl.ds(s * ROWS_T + t * CHUNK, nr)])

    plsc.subcore_barrier()

    def cnt_body(g, cy):
        pltpu.sync_copy(dst_hbm.at[wid * GCNT + g], dst_c)
        pltpu.sync_copy(
            oh_hbm.at[pl.ds((wid * GCNT + g) * CHUNK, CHUNK)], obuf)
        pltpu.sync_copy(obuf, cnt_sh.at[dst_c], add=True)
        return cy

    lax.fori_loop(0, GCNT, cnt_body, 0)

    plsc.subcore_barrier()

    for t, nr in ((0, CHUNK), (1, CHUNK), (2, CHUNK), (3, CHUNK),
                  (4, ROWS_T - 4 * CHUNK)):
        r0 = s * ROWS_T + t * CHUNK
        pltpu.sync_copy(cnt_sh.at[pl.ds(r0, nr)], obuf.at[pl.ds(0, nr)])
        pltpu.sync_copy(obuf.at[pl.ds(0, nr)],
                        cnt_hbm.at[pl.ds(c * NR + r0, nr)])


_sc_main = functools.partial(
    pl.kernel,
    out_type=jax.ShapeDtypeStruct((2 * NR, HALF), jnp.float32),
    mesh=plsc.VectorSubcoreMesh(core_axis_name="c", subcore_axis_name="s"),
    scratch_types=[
        pltpu.VMEM((4, CHUNK), jnp.int32),
        pltpu.VMEM((CHUNK, HALF), jnp.float32),
        pltpu.VMEM((CHUNK, HALF), jnp.float32),
        pltpu.VMEM_SHARED((NR, HALF), jnp.float32),
        pltpu.SemaphoreType.DMA,
        pltpu.SemaphoreType.DMA,
    ],
)(_sc_main_body)

_sc_cnt = functools.partial(
    pl.kernel,
    out_type=jax.ShapeDtypeStruct((2 * NR, HALF), jnp.float32),
    mesh=plsc.VectorSubcoreMesh(core_axis_name="c", subcore_axis_name="s"),
    scratch_types=[
        pltpu.VMEM((CHUNK,), jnp.int32),
        pltpu.VMEM((CHUNK, HALF), jnp.float32),
        pltpu.VMEM_SHARED((NR, HALF), jnp.float32),
        pltpu.SemaphoreType.DMA,
    ],
)(_sc_cnt_body)


@jax.jit
def kernel(x, edge_index, edge_attr, mask_node_indices, prelu_a, W_enc,
           emb1, emb2, emb3, emb4, W1, b1, W2, b2):
    mpad = jnp.pad(mask_node_indices, (0, MB - mask_node_indices.shape[0]),
                   constant_values=-1).reshape(1, MB)
    h_split = pl.pallas_call(
        _h_body,
        grid=(N // RB,),
        in_specs=[
            pl.BlockSpec((RB, D), lambda i: (i, 0)),
            pl.BlockSpec((D, D), lambda i: (0, 0)),
            pl.BlockSpec((1, 1), lambda i: (0, 0)),
            pl.BlockSpec((1, MB), lambda i: (0, 0)),
        ],
        out_specs=pl.BlockSpec((2, RB, HALF), lambda i: (0, i, 0)),
        out_shape=jax.ShapeDtypeStruct((2, N, HALF), jnp.float32),
    )(x, W_enc, prelu_a.reshape(1, 1), mpad)

    ea_p = jnp.pad(edge_attr, ((0, NPAD), (0, 0)))
    onehot = pl.pallas_call(
        _onehot_body,
        grid=(64,),
        in_specs=[pl.BlockSpec((EB, 4), lambda i: (i, 0))],
        out_specs=pl.BlockSpec((EB, HALF), lambda i: (i, 0)),
        out_shape=jax.ShapeDtypeStruct((EP, HALF), jnp.float32),
    )(ea_p)

    src_p = jnp.concatenate([edge_index[0], jnp.zeros((NPAD,), jnp.int32)])
    dst_p = jnp.concatenate(
        [edge_index[1], jnp.full((NPAD,), N, jnp.int32)])
    dst2d = dst_p.reshape(-1, CHUNK)
    src3d = jnp.concatenate(
        [src_p, src_p + N]).reshape(2, EP // CHUNK, CHUNK)
    sd2d = jnp.stack(
        [src3d,
         jnp.broadcast_to(dst2d[None], (2, EP // CHUNK, CHUNK))],
        axis=2).reshape(4 * (EP // CHUNK), CHUNK)
    h_flat = h_split.reshape(2 * N, HALF)
    z128 = jnp.zeros((CHUNK, HALF), jnp.float32)
    seg_flat = _sc_main(h_flat, sd2d, z128)
    cnt_flat = _sc_cnt(dst2d, onehot, z128, seg_flat)
    seg = seg_flat.reshape(2, NR, HALF)[:, :N, :]
    cnt = cnt_flat.reshape(2, NR, HALF)[:, :N, :]

    cemb = jnp.concatenate(
        [emb1, emb2, emb3, emb4, jnp.zeros((2, D), jnp.float32)], axis=0)
    out = pl.pallas_call(
        _mlp_body,
        grid=(N // RB,),
        in_specs=[
            pl.BlockSpec((2, RB, HALF), lambda i: (0, i, 0)),
            pl.BlockSpec((2, RB, HALF), lambda i: (0, i, 0)),
            pl.BlockSpec((2, RB, HALF), lambda i: (0, i, 0)),
            pl.BlockSpec((16, D), lambda i: (0, 0)),
            pl.BlockSpec((2 * D, D), lambda i: (0, 0)),
            pl.BlockSpec((1, 2 * D), lambda i: (0, 0)),
            pl.BlockSpec((OUT, 2 * D), lambda i: (0, 0)),
            pl.BlockSpec((1, OUT), lambda i: (0, 0)),
        ],
        out_specs=pl.BlockSpec((RB, OUT), lambda i: (i, 0)),
        out_shape=jax.ShapeDtypeStruct((N, OUT), jnp.float32),
    )(h_split, seg, cnt, cemb, W1, b1.reshape(1, 2 * D), W2,
      b2.reshape(1, OUT))
    return out
